# Initial kernel scaffold; baseline (speedup 1.0000x reference)
import jax
import jax.numpy as jnp
from jax.experimental import pallas as pl


def kernel(item_text_emb, gcn_item_emb, gcn_user_emb, users, user_hist, Wi1, bi1, Wi2, bi2, Wu1, bu1, Wu2, bu2):
    raise NotImplementedError("write your pallas kernel here")


# speedup vs baseline: 4.0031x; 4.0031x over previous
import functools

import jax
import jax.numpy as jnp
from jax import lax
from jax.experimental import pallas as pl
from jax.experimental.pallas import tpu as pltpu
from jax.experimental.pallas import tpu_sc as plsc

N_ITEMS = 100000
B = 4096
HIST = 200
D_TEXT = 128
D_GCN = 64
D_HID = 256
D_TOW = 128

ITEM_BLOCK = 2000


def _item_tower_body(xt_ref, xg_ref, w1a_ref, w1b_ref, b1_ref, w2_ref, b2_ref,
                     out_ref):
    h = jnp.dot(xt_ref[...], w1a_ref[...], preferred_element_type=jnp.float32)
    h += jnp.dot(xg_ref[...], w1b_ref[...], preferred_element_type=jnp.float32)
    h = jnp.maximum(h + b1_ref[...], 0.0)
    e = jnp.dot(h, w2_ref[...], preferred_element_type=jnp.float32) + b2_ref[...]
    n = jnp.sqrt(jnp.sum(e * e, axis=1, keepdims=True))
    out_ref[...] = e / jnp.maximum(n, 1e-12)


def _item_tower(item_text_emb, gcn_item_emb, w1a, w1b, b1, w2, b2):
    grid = N_ITEMS // ITEM_BLOCK
    return pl.pallas_call(
        _item_tower_body,
        grid=(grid,),
        in_specs=[
            pl.BlockSpec((ITEM_BLOCK, D_TEXT), lambda i: (i, 0)),
            pl.BlockSpec((ITEM_BLOCK, D_GCN), lambda i: (i, 0)),
            pl.BlockSpec((D_TEXT, D_HID), lambda i: (0, 0)),
            pl.BlockSpec((D_GCN, D_HID), lambda i: (0, 0)),
            pl.BlockSpec((1, D_HID), lambda i: (0, 0)),
            pl.BlockSpec((D_HID, D_TOW), lambda i: (0, 0)),
            pl.BlockSpec((1, D_TOW), lambda i: (0, 0)),
        ],
        out_specs=pl.BlockSpec((ITEM_BLOCK, D_TOW), lambda i: (i, 0)),
        out_shape=jax.ShapeDtypeStruct((N_ITEMS, D_TOW), jnp.float32),
    )(item_text_emb, gcn_item_emb, w1a, w1b, b1, w2, b2)


def _sc_pool_call(item_emb, gcn_user_emb, users, user_hist):
    info = plsc.get_sparse_core_info()
    nc, ns, nl = info.num_cores, info.num_subcores, info.num_lanes
    nw = nc * ns
    users_per_w = B // nw
    c0 = 128
    c1 = HIST - c0

    mesh = plsc.VectorSubcoreMesh(core_axis_name="c", subcore_axis_name="s")

    @functools.partial(
        pl.kernel,
        out_type=[
            jax.ShapeDtypeStruct((B, D_TOW), jnp.float32),
            jax.ShapeDtypeStruct((B, D_GCN), jnp.float32),
        ],
        mesh=mesh,
        compiler_params=pltpu.CompilerParams(use_tc_tiling_on_sc=False),
        scratch_types=[
            pltpu.VMEM((users_per_w,), jnp.int32),
            pltpu.VMEM((users_per_w, D_GCN), jnp.float32),
            pltpu.VMEM((c0,), jnp.int32),
            pltpu.VMEM((c1,), jnp.int32),
            pltpu.VMEM((c0, D_TOW), jnp.float32),
            pltpu.VMEM((c1, D_TOW), jnp.float32),
            pltpu.VMEM((D_TOW,), jnp.float32),
            pltpu.SemaphoreType.DMA,
        ],
    )
    def sc_pool(item_emb_h, ufeat_tab_h, users_h, hist_h, hist_out, ufeat_out,
                uid_v, ufeat_v, idx_a, idx_b, rows_a, rows_b, accrow_v, sem):
        wid = lax.axis_index("s") * nc + lax.axis_index("c")
        base = wid * users_per_w

        pltpu.sync_copy(users_h.at[pl.ds(base, users_per_w)], uid_v)
        pltpu.async_copy(ufeat_tab_h.at[uid_v], ufeat_v, sem).wait()
        pltpu.sync_copy(ufeat_v, ufeat_out.at[pl.ds(base, users_per_w)])

        nreg = D_TOW // nl

        def one_user(i, carry):
            b = base + i
            pltpu.sync_copy(hist_h.at[b, pl.ds(0, c0)], idx_a)
            pltpu.sync_copy(hist_h.at[b, pl.ds(c0, c1)], idx_b)
            pltpu.async_copy(item_emb_h.at[idx_a], rows_a, sem).wait()
            pltpu.async_copy(item_emb_h.at[idx_b], rows_b, sem).wait()

            def acc_a(r, acc):
                return tuple(acc[c] + rows_a[r, pl.ds(c * nl, nl)]
                             for c in range(nreg))

            def acc_b(r, acc):
                return tuple(acc[c] + rows_b[r, pl.ds(c * nl, nl)]
                             for c in range(nreg))

            zero = tuple(jnp.zeros((nl,), jnp.float32) for _ in range(nreg))
            acc = lax.fori_loop(0, c0, acc_a, zero)
            acc = lax.fori_loop(0, c1, acc_b, acc)
            for c in range(nreg):
                accrow_v[pl.ds(c * nl, nl)] = acc[c]
            pltpu.sync_copy(accrow_v, hist_out.at[b])
            return carry

        lax.fori_loop(0, users_per_w, one_user, 0)

    return sc_pool(item_emb, gcn_user_emb, users, user_hist)


def _user_tower_body(uf_ref, hs_ref, w1a_ref, w1b_ref, b1_ref, w2_ref, b2_ref,
                     out_ref):
    h = jnp.dot(uf_ref[...], w1a_ref[...], preferred_element_type=jnp.float32)
    h += jnp.dot(hs_ref[...], w1b_ref[...], preferred_element_type=jnp.float32)
    h = jnp.maximum(h + b1_ref[...], 0.0)
    e = jnp.dot(h, w2_ref[...], preferred_element_type=jnp.float32) + b2_ref[...]
    n = jnp.sqrt(jnp.sum(e * e, axis=1, keepdims=True))
    out_ref[...] = e / jnp.maximum(n, 1e-12)


def _user_tower(u_feat, hist_sum, w1a, w1b, b1, w2, b2):
    blk = 2048
    grid = B // blk
    return pl.pallas_call(
        _user_tower_body,
        grid=(grid,),
        in_specs=[
            pl.BlockSpec((blk, D_GCN), lambda i: (i, 0)),
            pl.BlockSpec((blk, D_TOW), lambda i: (i, 0)),
            pl.BlockSpec((D_GCN, D_HID), lambda i: (0, 0)),
            pl.BlockSpec((D_TOW, D_HID), lambda i: (0, 0)),
            pl.BlockSpec((1, D_HID), lambda i: (0, 0)),
            pl.BlockSpec((D_HID, D_TOW), lambda i: (0, 0)),
            pl.BlockSpec((1, D_TOW), lambda i: (0, 0)),
        ],
        out_specs=pl.BlockSpec((blk, D_TOW), lambda i: (i, 0)),
        out_shape=jax.ShapeDtypeStruct((B, D_TOW), jnp.float32),
    )(u_feat, hist_sum, w1a, w1b, b1, w2, b2)


def kernel(item_text_emb, gcn_item_emb, gcn_user_emb, users, user_hist,
           Wi1, bi1, Wi2, bi2, Wu1, bu1, Wu2, bu2):
    wi1a = Wi1[:D_TEXT]
    wi1b = Wi1[D_TEXT:]
    item_emb = _item_tower(item_text_emb, gcn_item_emb, wi1a, wi1b,
                           bi1.reshape(1, -1), Wi2, bi2.reshape(1, -1))

    users_i = users.astype(jnp.int32)
    hist_i = user_hist.astype(jnp.int32)
    hist_sum, u_feat = _sc_pool_call(item_emb, gcn_user_emb, users_i, hist_i)

    wu1a = Wu1[:D_GCN]
    wu1b = Wu1[D_GCN:] * (1.0 / HIST)
    return _user_tower(u_feat, hist_sum, wu1a, wu1b,
                       bu1.reshape(1, -1), Wu2, bu2.reshape(1, -1))


# speedup vs baseline: 7.5848x; 1.8947x over previous
import functools

import jax
import jax.numpy as jnp
from jax import lax
from jax.experimental import pallas as pl
from jax.experimental.pallas import tpu as pltpu
from jax.experimental.pallas import tpu_sc as plsc

N_ITEMS = 100000
B = 4096
HIST = 200
D_TEXT = 128
D_GCN = 64
D_HID = 256
D_TOW = 128

ITEM_BLOCK = 2000


def _item_tower_body(xt_ref, xg_ref, w1a_ref, w1b_ref, b1_ref, w2_ref, b2_ref,
                     out_ref):
    h = jnp.dot(xt_ref[...], w1a_ref[...], preferred_element_type=jnp.float32)
    h += jnp.dot(xg_ref[...], w1b_ref[...], preferred_element_type=jnp.float32)
    h = jnp.maximum(h + b1_ref[...], 0.0)
    e = jnp.dot(h, w2_ref[...], preferred_element_type=jnp.float32) + b2_ref[...]
    n = jnp.sqrt(jnp.sum(e * e, axis=1, keepdims=True))
    out_ref[...] = e / jnp.maximum(n, 1e-12)


def _item_tower(item_text_emb, gcn_item_emb, w1a, w1b, b1, w2, b2):
    grid = N_ITEMS // ITEM_BLOCK
    return pl.pallas_call(
        _item_tower_body,
        grid=(grid,),
        in_specs=[
            pl.BlockSpec((ITEM_BLOCK, D_TEXT), lambda i: (i, 0)),
            pl.BlockSpec((ITEM_BLOCK, D_GCN), lambda i: (i, 0)),
            pl.BlockSpec((D_TEXT, D_HID), lambda i: (0, 0)),
            pl.BlockSpec((D_GCN, D_HID), lambda i: (0, 0)),
            pl.BlockSpec((1, D_HID), lambda i: (0, 0)),
            pl.BlockSpec((D_HID, D_TOW), lambda i: (0, 0)),
            pl.BlockSpec((1, D_TOW), lambda i: (0, 0)),
        ],
        out_specs=pl.BlockSpec((ITEM_BLOCK, D_TOW), lambda i: (i, 0)),
        out_shape=jax.ShapeDtypeStruct((N_ITEMS, D_TOW), jnp.float32),
    )(item_text_emb, gcn_item_emb, w1a, w1b, b1, w2, b2)


def _sc_pool_call(item_emb, gcn_user_emb, users, user_hist):
    info = plsc.get_sparse_core_info()
    nc, ns, nl = info.num_cores, info.num_subcores, info.num_lanes
    nw = nc * ns
    upw = B // nw
    c0 = 128
    c1 = HIST - c0
    nreg = D_TOW // nl
    nslots = 2

    mesh = plsc.VectorSubcoreMesh(core_axis_name="c", subcore_axis_name="s")

    @functools.partial(
        pl.kernel,
        out_type=[
            jax.ShapeDtypeStruct((B, D_TOW), jnp.float32),
            jax.ShapeDtypeStruct((B, D_GCN), jnp.float32),
        ],
        mesh=mesh,
        compiler_params=pltpu.CompilerParams(use_tc_tiling_on_sc=False),
        scratch_types=[
            pltpu.VMEM((upw,), jnp.int32),
            pltpu.VMEM((upw, D_GCN), jnp.float32),
            pltpu.VMEM((upw, HIST), jnp.int32),
            pltpu.VMEM((nslots, HIST, D_TOW), jnp.float32),
            pltpu.VMEM((D_TOW,), jnp.float32),
            pltpu.SemaphoreType.DMA,
            pltpu.SemaphoreType.DMA,
            pltpu.SemaphoreType.DMA,
        ],
    )
    def sc_pool(item_h, utab_h, users_h, hist_h, hist_out, ufeat_out,
                uid_v, ufeat_v, idx_all, rows_v, accrow_v, sem0, sem1, semm):
        wid = lax.axis_index("s") * nc + lax.axis_index("c")
        base = wid * upw
        sems = (sem0, sem1)

        pltpu.sync_copy(hist_h.at[pl.ds(base, upw)], idx_all)

        def copies(slot, u):
            ra = rows_v.at[slot, pl.ds(0, c0)]
            rb = rows_v.at[slot, pl.ds(c0, c1)]
            sem = sems[slot]
            return (
                pltpu.make_async_copy(item_h.at[idx_all.at[u, pl.ds(0, c0)]], ra, sem),
                pltpu.make_async_copy(item_h.at[idx_all.at[u, pl.ds(c0, c1)]], rb, sem),
            )

        def issue(slot, u):
            for c in copies(slot, u):
                c.start()

        def wait_acc(slot, u):
            for c in copies(slot, u):
                c.wait()

            def acc_row(r, acc):
                return tuple(acc[k] + rows_v[slot, r, pl.ds(k * nl, nl)]
                             for k in range(nreg))

            zero = tuple(jnp.zeros((nl,), jnp.float32) for _ in range(nreg))
            acc = lax.fori_loop(0, HIST, acc_row, zero, unroll=2)
            for k in range(nreg):
                accrow_v[pl.ds(k * nl, nl)] = acc[k]
            pltpu.sync_copy(accrow_v, hist_out.at[base + u])

        issue(0, 0)
        issue(1, 1)

        pltpu.sync_copy(users_h.at[pl.ds(base, upw)], uid_v)
        pltpu.async_copy(utab_h.at[uid_v], ufeat_v, semm).wait()
        pltpu.sync_copy(ufeat_v, ufeat_out.at[pl.ds(base, upw)])

        def body(j, carry):
            u0 = 2 * j
            wait_acc(0, u0)

            @pl.when(u0 + 2 < upw)
            def _():
                issue(0, u0 + 2)

            wait_acc(1, u0 + 1)

            @pl.when(u0 + 3 < upw)
            def _():
                issue(1, u0 + 3)

            return carry

        lax.fori_loop(0, upw // 2, body, 0)

    return sc_pool(item_emb, gcn_user_emb, users, user_hist)


def _user_tower_body(uf_ref, hs_ref, w1a_ref, w1b_ref, b1_ref, w2_ref, b2_ref,
                     out_ref):
    h = jnp.dot(uf_ref[...], w1a_ref[...], preferred_element_type=jnp.float32)
    h += jnp.dot(hs_ref[...], w1b_ref[...], preferred_element_type=jnp.float32)
    h = jnp.maximum(h + b1_ref[...], 0.0)
    e = jnp.dot(h, w2_ref[...], preferred_element_type=jnp.float32) + b2_ref[...]
    n = jnp.sqrt(jnp.sum(e * e, axis=1, keepdims=True))
    out_ref[...] = e / jnp.maximum(n, 1e-12)


def _user_tower(u_feat, hist_sum, w1a, w1b, b1, w2, b2):
    blk = 2048
    grid = B // blk
    return pl.pallas_call(
        _user_tower_body,
        grid=(grid,),
        in_specs=[
            pl.BlockSpec((blk, D_GCN), lambda i: (i, 0)),
            pl.BlockSpec((blk, D_TOW), lambda i: (i, 0)),
            pl.BlockSpec((D_GCN, D_HID), lambda i: (0, 0)),
            pl.BlockSpec((D_TOW, D_HID), lambda i: (0, 0)),
            pl.BlockSpec((1, D_HID), lambda i: (0, 0)),
            pl.BlockSpec((D_HID, D_TOW), lambda i: (0, 0)),
            pl.BlockSpec((1, D_TOW), lambda i: (0, 0)),
        ],
        out_specs=pl.BlockSpec((blk, D_TOW), lambda i: (i, 0)),
        out_shape=jax.ShapeDtypeStruct((B, D_TOW), jnp.float32),
    )(u_feat, hist_sum, w1a, w1b, b1, w2, b2)


def kernel(item_text_emb, gcn_item_emb, gcn_user_emb, users, user_hist,
           Wi1, bi1, Wi2, bi2, Wu1, bu1, Wu2, bu2):
    wi1a = Wi1[:D_TEXT]
    wi1b = Wi1[D_TEXT:]
    item_emb = _item_tower(item_text_emb, gcn_item_emb, wi1a, wi1b,
                           bi1.reshape(1, -1), Wi2, bi2.reshape(1, -1))

    users_i = users.astype(jnp.int32)
    hist_i = user_hist.astype(jnp.int32)
    hist_sum, u_feat = _sc_pool_call(item_emb, gcn_user_emb, users_i, hist_i)

    wu1a = Wu1[:D_GCN]
    wu1b = Wu1[D_GCN:] * (1.0 / HIST)
    return _user_tower(u_feat, hist_sum, wu1a, wu1b,
                       bu1.reshape(1, -1), Wu2, bu2.reshape(1, -1))


# speedup vs baseline: 7.8640x; 1.0368x over previous
import functools

import jax
import jax.numpy as jnp
import numpy as np
from jax import lax
from jax.experimental import pallas as pl
from jax.experimental.pallas import tpu as pltpu
from jax.experimental.pallas import tpu_sc as plsc

N_ITEMS = 100000
B = 4096
HIST = 200
D_TEXT = 128
D_GCN = 64
D_HID = 256
D_TOW = 128

ITEM_BLOCK = 2000


def _item_tower_body(xt_ref, xg_ref, w1a_ref, w1b_ref, b1_ref, w2_ref, b2_ref,
                     out_ref):
    xt = xt_ref[...].astype(jnp.bfloat16)
    xg = xg_ref[...].astype(jnp.bfloat16)
    h = jnp.dot(xt, w1a_ref[...], preferred_element_type=jnp.float32)
    h += jnp.dot(xg, w1b_ref[...], preferred_element_type=jnp.float32)
    h = jnp.maximum(h + b1_ref[...], 0.0).astype(jnp.bfloat16)
    e = jnp.dot(h, w2_ref[...], preferred_element_type=jnp.float32) + b2_ref[...]
    n = jnp.sqrt(jnp.sum(e * e, axis=1, keepdims=True))
    out_ref[...] = (e / jnp.maximum(n, 1e-12)).astype(jnp.bfloat16)


def _item_tower(item_text_emb, gcn_item_emb, w1a, w1b, b1, w2, b2):
    grid = N_ITEMS // ITEM_BLOCK
    return pl.pallas_call(
        _item_tower_body,
        grid=(grid,),
        in_specs=[
            pl.BlockSpec((ITEM_BLOCK, D_TEXT), lambda i: (i, 0)),
            pl.BlockSpec((ITEM_BLOCK, D_GCN), lambda i: (i, 0)),
            pl.BlockSpec((D_TEXT, D_HID), lambda i: (0, 0)),
            pl.BlockSpec((D_GCN, D_HID), lambda i: (0, 0)),
            pl.BlockSpec((1, D_HID), lambda i: (0, 0)),
            pl.BlockSpec((D_HID, D_TOW), lambda i: (0, 0)),
            pl.BlockSpec((1, D_TOW), lambda i: (0, 0)),
        ],
        out_specs=pl.BlockSpec((ITEM_BLOCK, D_TOW), lambda i: (i, 0)),
        out_shape=jax.ShapeDtypeStruct((N_ITEMS, D_TOW), jnp.bfloat16),
    )(item_text_emb, gcn_item_emb, w1a, w1b, b1, w2, b2)


def _sc_pool_call(item_emb, gcn_user_emb, users, user_hist):
    info = plsc.get_sparse_core_info()
    nc, ns, nl = info.num_cores, info.num_subcores, info.num_lanes
    nw = nc * ns
    upw = B // nw
    c0 = 128
    c1 = HIST - c0
    npair = D_TOW // (2 * nl)
    nslots = 4

    mesh = plsc.VectorSubcoreMesh(core_axis_name="c", subcore_axis_name="s")

    @functools.partial(
        pl.kernel,
        out_type=[
            jax.ShapeDtypeStruct((B, D_TOW), jnp.float32),
            jax.ShapeDtypeStruct((B, D_GCN), jnp.float32),
        ],
        mesh=mesh,
        compiler_params=pltpu.CompilerParams(use_tc_tiling_on_sc=False,
                                             needs_layout_passes=False),
        scratch_types=[
            pltpu.VMEM((upw,), jnp.int32),
            pltpu.VMEM((upw, D_GCN), jnp.float32),
            pltpu.VMEM((upw, HIST), jnp.int32),
            pltpu.VMEM((nslots, HIST, D_TOW), jnp.bfloat16),
            pltpu.VMEM((D_TOW,), jnp.float32),
            pltpu.SemaphoreType.DMA,
            pltpu.SemaphoreType.DMA,
            pltpu.SemaphoreType.DMA,
            pltpu.SemaphoreType.DMA,
            pltpu.SemaphoreType.DMA,
        ],
    )
    def sc_pool(item_h, utab_h, users_h, hist_h, hist_out, ufeat_out,
                uid_v, ufeat_v, idx_all, rows_v, accrow_v,
                sem0, sem1, sem2, sem3, semm):
        wid = lax.axis_index("s") * nc + lax.axis_index("c")
        base = wid * upw
        sems = (sem0, sem1, sem2, sem3)

        pltpu.sync_copy(hist_h.at[pl.ds(base, upw)], idx_all)

        def copies(slot, u):
            ra = rows_v.at[slot, pl.ds(0, c0)]
            rb = rows_v.at[slot, pl.ds(c0, c1)]
            sem = sems[slot]
            return (
                pltpu.make_async_copy(item_h.at[idx_all.at[u, pl.ds(0, c0)]], ra, sem),
                pltpu.make_async_copy(item_h.at[idx_all.at[u, pl.ds(c0, c1)]], rb, sem),
            )

        def issue(slot, u):
            for c in copies(slot, u):
                c.start()

        def wait_acc(slot, u):
            for c in copies(slot, u):
                c.wait()

            def acc_row(r, acc):
                new = []
                for k in range(npair):
                    x = rows_v[slot, r, pl.ds(k * 2 * nl, 2 * nl)]
                    lo, hi = plsc.unpack(x, format=plsc.PackFormat.INTERLEAVED)
                    new.append(acc[2 * k] + lo)
                    new.append(acc[2 * k + 1] + hi)
                return tuple(new)

            zero = tuple(jnp.zeros((nl,), jnp.float32)
                         for _ in range(2 * npair))
            acc = lax.fori_loop(0, HIST, acc_row, zero, unroll=2)
            for k in range(2 * npair):
                accrow_v[pl.ds(k * nl, nl)] = acc[k]
            pltpu.sync_copy(accrow_v, hist_out.at[base + u])

        for s in range(nslots):
            issue(s, s)

        pltpu.sync_copy(users_h.at[pl.ds(base, upw)], uid_v)
        pltpu.async_copy(utab_h.at[uid_v], ufeat_v, semm).wait()
        pltpu.sync_copy(ufeat_v, ufeat_out.at[pl.ds(base, upw)])

        def body(j, carry):
            u0 = nslots * j
            for s in range(nslots):
                wait_acc(s, u0 + s)

                @pl.when(u0 + s + nslots < upw)
                def _(s=s):
                    issue(s, u0 + s + nslots)

            return carry

        lax.fori_loop(0, upw // nslots, body, 0)

    return sc_pool(item_emb, gcn_user_emb, users, user_hist)


def _user_tower_body(uf_ref, hs_ref, w1a_ref, w1b_ref, b1_ref, w2_ref, b2_ref,
                     out_ref):
    h = jnp.dot(uf_ref[...], w1a_ref[...], preferred_element_type=jnp.float32)
    h += jnp.dot(hs_ref[...], w1b_ref[...], preferred_element_type=jnp.float32)
    h = jnp.maximum(h + b1_ref[...], 0.0)
    e = jnp.dot(h, w2_ref[...], preferred_element_type=jnp.float32) + b2_ref[...]
    n = jnp.sqrt(jnp.sum(e * e, axis=1, keepdims=True))
    out_ref[...] = e / jnp.maximum(n, 1e-12)


def _user_tower(u_feat, hist_sum, w1a, w1b, b1, w2, b2):
    blk = 2048
    grid = B // blk
    return pl.pallas_call(
        _user_tower_body,
        grid=(grid,),
        in_specs=[
            pl.BlockSpec((blk, D_GCN), lambda i: (i, 0)),
            pl.BlockSpec((blk, D_TOW), lambda i: (i, 0)),
            pl.BlockSpec((D_GCN, D_HID), lambda i: (0, 0)),
            pl.BlockSpec((D_TOW, D_HID), lambda i: (0, 0)),
            pl.BlockSpec((1, D_HID), lambda i: (0, 0)),
            pl.BlockSpec((D_HID, D_TOW), lambda i: (0, 0)),
            pl.BlockSpec((1, D_TOW), lambda i: (0, 0)),
        ],
        out_specs=pl.BlockSpec((blk, D_TOW), lambda i: (i, 0)),
        out_shape=jax.ShapeDtypeStruct((B, D_TOW), jnp.float32),
    )(u_feat, hist_sum, w1a, w1b, b1, w2, b2)


_UNPACK_PERM = np.concatenate(
    [np.concatenate([32 * g + 2 * np.arange(16),
                     32 * g + 2 * np.arange(16) + 1]) for g in range(D_TOW // 32)])


def kernel(item_text_emb, gcn_item_emb, gcn_user_emb, users, user_hist,
           Wi1, bi1, Wi2, bi2, Wu1, bu1, Wu2, bu2):
    wi1a = Wi1[:D_TEXT].astype(jnp.bfloat16)
    wi1b = Wi1[D_TEXT:].astype(jnp.bfloat16)
    item_emb = _item_tower(item_text_emb, gcn_item_emb, wi1a, wi1b,
                           bi1.reshape(1, -1), Wi2.astype(jnp.bfloat16),
                           bi2.reshape(1, -1))

    users_i = users.astype(jnp.int32)
    hist_i = user_hist.astype(jnp.int32)
    hist_sum, u_feat = _sc_pool_call(item_emb, gcn_user_emb, users_i, hist_i)

    wu1a = Wu1[:D_GCN]
    wu1b = Wu1[D_GCN:][_UNPACK_PERM] * (1.0 / HIST)
    return _user_tower(u_feat, hist_sum, wu1a, wu1b,
                       bu1.reshape(1, -1), Wu2, bu2.reshape(1, -1))


# speedup vs baseline: 8.9292x; 1.1354x over previous
import functools

import jax
import jax.numpy as jnp
import numpy as np
from jax import lax
from jax.experimental import pallas as pl
from jax.experimental.pallas import tpu as pltpu
from jax.experimental.pallas import tpu_sc as plsc

N_ITEMS = 100000
B = 4096
HIST = 200
D_TEXT = 128
D_GCN = 64
D_HID = 256
D_TOW = 128

ITEM_BLOCK = 2000


def _item_tower_body(xt_ref, xg_ref, w1a_ref, w1b_ref, b1_ref, w2_ref, b2_ref,
                     out_ref):
    xt = xt_ref[...].astype(jnp.bfloat16)
    xg = xg_ref[...].astype(jnp.bfloat16)
    h = jnp.dot(xt, w1a_ref[...], preferred_element_type=jnp.float32)
    h += jnp.dot(xg, w1b_ref[...], preferred_element_type=jnp.float32)
    h = jnp.maximum(h + b1_ref[...], 0.0).astype(jnp.bfloat16)
    e = jnp.dot(h, w2_ref[...], preferred_element_type=jnp.float32) + b2_ref[...]
    n = jnp.sqrt(jnp.sum(e * e, axis=1, keepdims=True))
    e = e / jnp.maximum(n, 1e-12)
    eb = lax.bitcast_convert_type(e.astype(jnp.bfloat16), jnp.uint16)
    lo = eb[:, :D_TOW // 2].astype(jnp.uint32)
    hi = eb[:, D_TOW // 2:].astype(jnp.uint32)
    packed = lax.bitcast_convert_type(lo | (hi << 16), jnp.float32)
    out_ref[:, :D_TOW // 2] = packed
    out_ref[:, D_TOW // 2:] = jnp.zeros_like(packed)


def _item_tower(item_text_emb, gcn_item_emb, w1a, w1b, b1, w2, b2):
    grid = N_ITEMS // ITEM_BLOCK
    return pl.pallas_call(
        _item_tower_body,
        grid=(grid,),
        in_specs=[
            pl.BlockSpec((ITEM_BLOCK, D_TEXT), lambda i: (i, 0)),
            pl.BlockSpec((ITEM_BLOCK, D_GCN), lambda i: (i, 0)),
            pl.BlockSpec((D_TEXT, D_HID), lambda i: (0, 0)),
            pl.BlockSpec((D_GCN, D_HID), lambda i: (0, 0)),
            pl.BlockSpec((1, D_HID), lambda i: (0, 0)),
            pl.BlockSpec((D_HID, D_TOW), lambda i: (0, 0)),
            pl.BlockSpec((1, D_TOW), lambda i: (0, 0)),
        ],
        out_specs=pl.BlockSpec((ITEM_BLOCK, D_TOW), lambda i: (i, 0)),
        out_shape=jax.ShapeDtypeStruct((N_ITEMS, D_TOW), jnp.float32),
    )(item_text_emb, gcn_item_emb, w1a, w1b, b1, w2, b2)


def _sc_pool_call(item_tab, gcn_user_emb, users, user_hist_flat):
    info = plsc.get_sparse_core_info()
    nc, ns, nl = info.num_cores, info.num_subcores, info.num_lanes
    nw = nc * ns
    upw = B // nw
    c0 = 128
    c1 = HIST - c0
    dpack = D_TOW // 2
    npair = dpack // nl
    nslots = 4

    mesh = plsc.VectorSubcoreMesh(core_axis_name="c", subcore_axis_name="s")

    @functools.partial(
        pl.kernel,
        out_type=[
            jax.ShapeDtypeStruct((B, D_TOW), jnp.float32),
            jax.ShapeDtypeStruct((B, D_GCN), jnp.float32),
        ],
        mesh=mesh,
        compiler_params=pltpu.CompilerParams(use_tc_tiling_on_sc=False,
                                             needs_layout_passes=False),
        scratch_types=[
            pltpu.VMEM((upw,), jnp.int32),
            pltpu.VMEM((upw, D_GCN), jnp.float32),
            pltpu.VMEM((upw * HIST,), jnp.int32),
            pltpu.VMEM((nslots, HIST, dpack), jnp.float32),
            pltpu.VMEM((D_TOW,), jnp.float32),
            pltpu.SemaphoreType.DMA,
            pltpu.SemaphoreType.DMA,
            pltpu.SemaphoreType.DMA,
            pltpu.SemaphoreType.DMA,
            pltpu.SemaphoreType.DMA,
        ],
    )
    def sc_pool(item_h, utab_h, users_h, hist_h, hist_out, ufeat_out,
                uid_v, ufeat_v, idx_all, rows_v, accrow_v,
                sem0, sem1, sem2, sem3, semm):
        wid = lax.axis_index("s") * nc + lax.axis_index("c")
        base = wid * upw
        sems = (sem0, sem1, sem2, sem3)

        pltpu.sync_copy(hist_h.at[pl.ds(base * HIST, upw * HIST)], idx_all)

        def copies(slot, u):
            ra = rows_v.at[slot, pl.ds(0, c0)]
            rb = rows_v.at[slot, pl.ds(c0, c1)]
            sem = sems[slot]
            return (
                pltpu.make_async_copy(
                    item_h.at[idx_all.at[pl.ds(u * HIST, c0)]], ra, sem),
                pltpu.make_async_copy(
                    item_h.at[idx_all.at[pl.ds(u * HIST + c0, c1)]], rb, sem),
            )

        def issue(slot, u):
            for c in copies(slot, u):
                c.start()

        def wait_acc(slot, u):
            for c in copies(slot, u):
                c.wait()

            def acc_row(r, acc):
                new = list(acc)
                for k in range(npair):
                    x = rows_v[slot, r, pl.ds(k * nl, nl)]
                    lo, hi = plsc.unpack(plsc.bitcast(x, jnp.bfloat16),
                                         format=plsc.PackFormat.INTERLEAVED)
                    new[k] = acc[k] + lo
                    new[npair + k] = acc[npair + k] + hi
                return tuple(new)

            zero = tuple(jnp.zeros((nl,), jnp.float32)
                         for _ in range(2 * npair))
            acc = lax.fori_loop(0, HIST, acc_row, zero, unroll=2)
            for k in range(2 * npair):
                accrow_v[pl.ds(k * nl, nl)] = acc[k]
            pltpu.sync_copy(accrow_v, hist_out.at[base + u])

        for s in range(nslots):
            issue(s, s)

        pltpu.sync_copy(users_h.at[pl.ds(base, upw)], uid_v)
        pltpu.async_copy(utab_h.at[uid_v], ufeat_v, semm).wait()
        pltpu.sync_copy(ufeat_v, ufeat_out.at[pl.ds(base, upw)])

        def body(j, carry):
            u0 = nslots * j
            for s in range(nslots):
                wait_acc(s, u0 + s)

                @pl.when(u0 + s + nslots < upw)
                def _(s=s):
                    issue(s, u0 + s + nslots)

            return carry

        lax.fori_loop(0, upw // nslots, body, 0)

    return sc_pool(item_tab, gcn_user_emb, users, user_hist_flat)


def _user_tower_body(uf_ref, hs_ref, w1a_ref, w1b_ref, b1_ref, w2_ref, b2_ref,
                     out_ref):
    h = jnp.dot(uf_ref[...], w1a_ref[...], preferred_element_type=jnp.float32)
    h += jnp.dot(hs_ref[...], w1b_ref[...], preferred_element_type=jnp.float32)
    h = jnp.maximum(h + b1_ref[...], 0.0)
    e = jnp.dot(h, w2_ref[...], preferred_element_type=jnp.float32) + b2_ref[...]
    n = jnp.sqrt(jnp.sum(e * e, axis=1, keepdims=True))
    out_ref[...] = e / jnp.maximum(n, 1e-12)


def _user_tower(u_feat, hist_sum, w1a, w1b, b1, w2, b2):
    blk = 2048
    grid = B // blk
    return pl.pallas_call(
        _user_tower_body,
        grid=(grid,),
        in_specs=[
            pl.BlockSpec((blk, D_GCN), lambda i: (i, 0)),
            pl.BlockSpec((blk, D_TOW), lambda i: (i, 0)),
            pl.BlockSpec((D_GCN, D_HID), lambda i: (0, 0)),
            pl.BlockSpec((D_TOW, D_HID), lambda i: (0, 0)),
            pl.BlockSpec((1, D_HID), lambda i: (0, 0)),
            pl.BlockSpec((D_HID, D_TOW), lambda i: (0, 0)),
            pl.BlockSpec((1, D_TOW), lambda i: (0, 0)),
        ],
        out_specs=pl.BlockSpec((blk, D_TOW), lambda i: (i, 0)),
        out_shape=jax.ShapeDtypeStruct((B, D_TOW), jnp.float32),
    )(u_feat, hist_sum, w1a, w1b, b1, w2, b2)


def kernel(item_text_emb, gcn_item_emb, gcn_user_emb, users, user_hist,
           Wi1, bi1, Wi2, bi2, Wu1, bu1, Wu2, bu2):
    wi1a = Wi1[:D_TEXT].astype(jnp.bfloat16)
    wi1b = Wi1[D_TEXT:].astype(jnp.bfloat16)
    item_emb = _item_tower(item_text_emb, gcn_item_emb, wi1a, wi1b,
                           bi1.reshape(1, -1), Wi2.astype(jnp.bfloat16),
                           bi2.reshape(1, -1))
    item_tab = item_emb.reshape(2 * N_ITEMS, D_TOW // 2)

    users_i = users.astype(jnp.int32)
    hist_i = (user_hist.astype(jnp.int32) * 2).reshape(-1)
    hist_sum, u_feat = _sc_pool_call(item_tab, gcn_user_emb, users_i, hist_i)

    wu1a = Wu1[:D_GCN]
    wu1b = Wu1[D_GCN:] * (1.0 / HIST)
    return _user_tower(u_feat, hist_sum, wu1a, wu1b,
                       bu1.reshape(1, -1), Wu2, bu2.reshape(1, -1))


# speedup vs baseline: 9.9996x; 1.1199x over previous
import functools

import jax
import jax.numpy as jnp
import numpy as np
from jax import lax
from jax.experimental import pallas as pl
from jax.experimental.pallas import tpu as pltpu
from jax.experimental.pallas import tpu_sc as plsc

N_ITEMS = 100000
B = 4096
HIST = 200
D_TEXT = 128
D_GCN = 64
D_HID = 256
D_TOW = 128

ITEM_BLOCK = 6400


def _item_tower_body(xt_ref, xgt_ref, w1a_ref, w1b_ref, b1_ref, w2_ref, b2_ref,
                     out_ref):
    xt = xt_ref[...].astype(jnp.bfloat16)
    xgt = xgt_ref[...].astype(jnp.bfloat16)
    h = jnp.dot(xt, w1a_ref[...], preferred_element_type=jnp.float32)
    h += lax.dot_general(xgt, w1b_ref[...], (((0,), (0,)), ((), ())),
                         preferred_element_type=jnp.float32)
    h = jnp.maximum(h + b1_ref[...], 0.0).astype(jnp.bfloat16)
    e = jnp.dot(h, w2_ref[...], preferred_element_type=jnp.float32) + b2_ref[...]
    n = jnp.sqrt(jnp.sum(e * e, axis=1, keepdims=True))
    e = e / jnp.maximum(n, 1e-12)
    eb = lax.bitcast_convert_type(e.astype(jnp.bfloat16), jnp.uint16)
    lo = eb[:, :D_TOW // 2].astype(jnp.uint32)
    hi = eb[:, D_TOW // 2:].astype(jnp.uint32)
    packed = lax.bitcast_convert_type(lo | (hi << 16), jnp.float32)
    out_ref[:, :D_TOW // 2] = packed


def _item_tower(item_text_emb, gcn_item_emb_t, w1a, w1b, b1, w2, b2):
    grid = (N_ITEMS + ITEM_BLOCK - 1) // ITEM_BLOCK
    return pl.pallas_call(
        _item_tower_body,
        grid=(grid,),
        in_specs=[
            pl.BlockSpec((ITEM_BLOCK, D_TEXT), lambda i: (i, 0)),
            pl.BlockSpec((D_GCN, ITEM_BLOCK), lambda i: (0, i)),
            pl.BlockSpec((D_TEXT, D_HID), lambda i: (0, 0)),
            pl.BlockSpec((D_GCN, D_HID), lambda i: (0, 0)),
            pl.BlockSpec((1, D_HID), lambda i: (0, 0)),
            pl.BlockSpec((D_HID, D_TOW), lambda i: (0, 0)),
            pl.BlockSpec((1, D_TOW), lambda i: (0, 0)),
        ],
        out_specs=pl.BlockSpec((ITEM_BLOCK, D_TOW), lambda i: (i, 0)),
        out_shape=jax.ShapeDtypeStruct((N_ITEMS, D_TOW), jnp.float32),
    )(item_text_emb, gcn_item_emb_t, w1a, w1b, b1, w2, b2)


def _sc_pool_call(item_tab, gcn_user_emb, users, user_hist_flat):
    info = plsc.get_sparse_core_info()
    nc, ns, nl = info.num_cores, info.num_subcores, info.num_lanes
    nw = nc * ns
    upw = B // nw
    c0 = 128
    c1 = HIST - c0
    dpack = D_TOW // 2
    npair = dpack // nl
    nslots = 4

    mesh = plsc.VectorSubcoreMesh(core_axis_name="c", subcore_axis_name="s")

    @functools.partial(
        pl.kernel,
        out_type=[
            jax.ShapeDtypeStruct((B, D_TOW), jnp.float32),
            jax.ShapeDtypeStruct((B, D_GCN), jnp.float32),
        ],
        mesh=mesh,
        compiler_params=pltpu.CompilerParams(use_tc_tiling_on_sc=False,
                                             needs_layout_passes=False),
        scratch_types=[
            pltpu.VMEM((upw,), jnp.int32),
            pltpu.VMEM((upw, D_GCN), jnp.float32),
            pltpu.VMEM((upw * HIST,), jnp.int32),
            pltpu.VMEM((nslots, HIST, dpack), jnp.float32),
            pltpu.VMEM((D_TOW,), jnp.float32),
            pltpu.SemaphoreType.DMA,
            pltpu.SemaphoreType.DMA,
            pltpu.SemaphoreType.DMA,
            pltpu.SemaphoreType.DMA,
            pltpu.SemaphoreType.DMA,
        ],
    )
    def sc_pool(item_h, utab_h, users_h, hist_h, hist_out, ufeat_out,
                uid_v, ufeat_v, idx_all, rows_v, accrow_v,
                sem0, sem1, sem2, sem3, semm):
        wid = lax.axis_index("s") * nc + lax.axis_index("c")
        base = wid * upw
        sems = (sem0, sem1, sem2, sem3)

        pltpu.sync_copy(hist_h.at[pl.ds(base * HIST, upw * HIST)], idx_all)

        def copies(slot, u):
            ra = rows_v.at[slot, pl.ds(0, c0)]
            rb = rows_v.at[slot, pl.ds(c0, c1)]
            sem = sems[slot]
            return (
                pltpu.make_async_copy(
                    item_h.at[idx_all.at[pl.ds(u * HIST, c0)]], ra, sem),
                pltpu.make_async_copy(
                    item_h.at[idx_all.at[pl.ds(u * HIST + c0, c1)]], rb, sem),
            )

        def issue(slot, u):
            for c in copies(slot, u):
                c.start()

        def wait_acc(slot, u):
            for c in copies(slot, u):
                c.wait()

            def acc_row(r, acc):
                new = list(acc)
                for k in range(npair):
                    x = rows_v[slot, r, pl.ds(k * nl, nl)]
                    lo, hi = plsc.unpack(plsc.bitcast(x, jnp.bfloat16),
                                         format=plsc.PackFormat.INTERLEAVED)
                    new[k] = acc[k] + lo
                    new[npair + k] = acc[npair + k] + hi
                return tuple(new)

            zero = tuple(jnp.zeros((nl,), jnp.float32)
                         for _ in range(2 * npair))
            acc = lax.fori_loop(0, HIST, acc_row, zero, unroll=2)
            for k in range(2 * npair):
                accrow_v[pl.ds(k * nl, nl)] = acc[k]
            pltpu.sync_copy(accrow_v, hist_out.at[base + u])

        for s in range(nslots):
            issue(s, s)

        pltpu.sync_copy(users_h.at[pl.ds(base, upw)], uid_v)
        pltpu.async_copy(utab_h.at[uid_v], ufeat_v, semm).wait()
        pltpu.sync_copy(ufeat_v, ufeat_out.at[pl.ds(base, upw)])

        def body(j, carry):
            u0 = nslots * j
            for s in range(nslots):
                wait_acc(s, u0 + s)

                @pl.when(u0 + s + nslots < upw)
                def _(s=s):
                    issue(s, u0 + s + nslots)

            return carry

        lax.fori_loop(0, upw // nslots, body, 0)

    return sc_pool(item_tab, gcn_user_emb, users, user_hist_flat)


def _user_tower_body(uf_ref, hs_ref, w1a_ref, w1b_ref, b1_ref, w2_ref, b2_ref,
                     out_ref):
    h = jnp.dot(uf_ref[...], w1a_ref[...], preferred_element_type=jnp.float32)
    h += jnp.dot(hs_ref[...], w1b_ref[...], preferred_element_type=jnp.float32)
    h = jnp.maximum(h + b1_ref[...], 0.0)
    e = jnp.dot(h, w2_ref[...], preferred_element_type=jnp.float32) + b2_ref[...]
    n = jnp.sqrt(jnp.sum(e * e, axis=1, keepdims=True))
    out_ref[...] = e / jnp.maximum(n, 1e-12)


def _user_tower(u_feat, hist_sum, w1a, w1b, b1, w2, b2):
    blk = 2048
    grid = B // blk
    return pl.pallas_call(
        _user_tower_body,
        grid=(grid,),
        in_specs=[
            pl.BlockSpec((blk, D_GCN), lambda i: (i, 0)),
            pl.BlockSpec((blk, D_TOW), lambda i: (i, 0)),
            pl.BlockSpec((D_GCN, D_HID), lambda i: (0, 0)),
            pl.BlockSpec((D_TOW, D_HID), lambda i: (0, 0)),
            pl.BlockSpec((1, D_HID), lambda i: (0, 0)),
            pl.BlockSpec((D_HID, D_TOW), lambda i: (0, 0)),
            pl.BlockSpec((1, D_TOW), lambda i: (0, 0)),
        ],
        out_specs=pl.BlockSpec((blk, D_TOW), lambda i: (i, 0)),
        out_shape=jax.ShapeDtypeStruct((B, D_TOW), jnp.float32),
    )(u_feat, hist_sum, w1a, w1b, b1, w2, b2)


def kernel(item_text_emb, gcn_item_emb, gcn_user_emb, users, user_hist,
           Wi1, bi1, Wi2, bi2, Wu1, bu1, Wu2, bu2):
    wi1a = Wi1[:D_TEXT].astype(jnp.bfloat16)
    wi1b = Wi1[D_TEXT:].astype(jnp.bfloat16)
    item_emb = _item_tower(item_text_emb, gcn_item_emb.T, wi1a, wi1b,
                           bi1.reshape(1, -1), Wi2.astype(jnp.bfloat16),
                           bi2.reshape(1, -1))
    item_tab = item_emb.reshape(2 * N_ITEMS, D_TOW // 2)

    users_i = users.astype(jnp.int32)
    hist_i = (user_hist.astype(jnp.int32) * 2).reshape(-1)
    hist_sum, u_feat = _sc_pool_call(item_tab, gcn_user_emb, users_i, hist_i)

    wu1a = Wu1[:D_GCN]
    wu1b = Wu1[D_GCN:] * (1.0 / HIST)
    return _user_tower(u_feat, hist_sum, wu1a, wu1b,
                       bu1.reshape(1, -1), Wu2, bu2.reshape(1, -1))


# speedup vs baseline: 11.1450x; 1.1145x over previous
import functools

import jax
import jax.numpy as jnp
import numpy as np
from jax import lax
from jax.experimental import pallas as pl
from jax.experimental.pallas import tpu as pltpu
from jax.experimental.pallas import tpu_sc as plsc

N_ITEMS = 100000
N_USERS = 100000
B = 4096
HIST = 200
D_TEXT = 128
D_GCN = 64
D_HID = 256
D_TOW = 128

ITEM_BLOCK = 6400


def _item_tower_body(xt_ref, xgt_ref, w1a_ref, w1b_ref, b1_ref, w2_ref, b2_ref,
                     out_ref):
    xt = xt_ref[...].astype(jnp.bfloat16)
    xgt = xgt_ref[...].astype(jnp.bfloat16)
    h = jnp.dot(xt, w1a_ref[...], preferred_element_type=jnp.float32)
    h += lax.dot_general(xgt, w1b_ref[...], (((0,), (0,)), ((), ())),
                         preferred_element_type=jnp.float32)
    h = jnp.maximum(h + b1_ref[...], 0.0).astype(jnp.bfloat16)
    e = jnp.dot(h, w2_ref[...], preferred_element_type=jnp.float32) + b2_ref[...]
    n = jnp.sqrt(jnp.sum(e * e, axis=1, keepdims=True))
    e = e / jnp.maximum(n, 1e-12)
    eb = lax.bitcast_convert_type(e.astype(jnp.bfloat16), jnp.uint16)
    lo = eb[:, :D_TOW // 2].astype(jnp.uint32)
    hi = eb[:, D_TOW // 2:].astype(jnp.uint32)
    packed = lax.bitcast_convert_type(lo | (hi << 16), jnp.float32)
    out_ref[:, :D_TOW // 2] = packed


def _item_tower(item_text_emb, gcn_item_emb_t, w1a, w1b, b1, w2, b2):
    grid = (N_ITEMS + ITEM_BLOCK - 1) // ITEM_BLOCK
    return pl.pallas_call(
        _item_tower_body,
        grid=(grid,),
        in_specs=[
            pl.BlockSpec((ITEM_BLOCK, D_TEXT), lambda i: (i, 0)),
            pl.BlockSpec((D_GCN, ITEM_BLOCK), lambda i: (0, i)),
            pl.BlockSpec((D_TEXT, D_HID), lambda i: (0, 0)),
            pl.BlockSpec((D_GCN, D_HID), lambda i: (0, 0)),
            pl.BlockSpec((1, D_HID), lambda i: (0, 0)),
            pl.BlockSpec((D_HID, D_TOW), lambda i: (0, 0)),
            pl.BlockSpec((1, D_TOW), lambda i: (0, 0)),
        ],
        out_specs=pl.BlockSpec((ITEM_BLOCK, D_TOW), lambda i: (i, 0)),
        out_shape=jax.ShapeDtypeStruct((N_ITEMS, D_TOW), jnp.float32),
    )(item_text_emb, gcn_item_emb_t, w1a, w1b, b1, w2, b2)


def _user_pre_body(xgt_ref, w_ref, b_ref, out_ref):
    xgt = xgt_ref[...].astype(jnp.bfloat16)
    g = lax.dot_general(xgt, w_ref[...], (((0,), (0,)), ((), ())),
                        preferred_element_type=jnp.float32) + b_ref[...]
    gb = lax.bitcast_convert_type(g.astype(jnp.bfloat16), jnp.uint16)
    lo = gb[:, :D_HID // 2].astype(jnp.uint32)
    hi = gb[:, D_HID // 2:].astype(jnp.uint32)
    out_ref[...] = lax.bitcast_convert_type(lo | (hi << 16), jnp.float32)


def _user_pre(gcn_user_emb_t, w1a, b1):
    blk = 12800
    grid = (N_USERS + blk - 1) // blk
    return pl.pallas_call(
        _user_pre_body,
        grid=(grid,),
        in_specs=[
            pl.BlockSpec((D_GCN, blk), lambda i: (0, i)),
            pl.BlockSpec((D_GCN, D_HID), lambda i: (0, 0)),
            pl.BlockSpec((1, D_HID), lambda i: (0, 0)),
        ],
        out_specs=pl.BlockSpec((blk, D_HID // 2), lambda i: (i, 0)),
        out_shape=jax.ShapeDtypeStruct((N_USERS, D_HID // 2), jnp.float32),
    )(gcn_user_emb_t, w1a, b1)


def _sc_ufeat_call(g_tab, users):
    info = plsc.get_sparse_core_info()
    nc, ns, nl = info.num_cores, info.num_subcores, info.num_lanes
    nw = nc * ns
    upw = B // nw

    mesh = plsc.VectorSubcoreMesh(core_axis_name="c", subcore_axis_name="s")

    @functools.partial(
        pl.kernel,
        out_type=jax.ShapeDtypeStruct((B, D_HID // 2), jnp.float32),
        mesh=mesh,
        compiler_params=pltpu.CompilerParams(use_tc_tiling_on_sc=False),
        scratch_types=[
            pltpu.VMEM((upw,), jnp.int32),
            pltpu.VMEM((upw, D_HID // 2), jnp.float32),
            pltpu.SemaphoreType.DMA,
        ],
    )
    def sc_ufeat(g_h, users_h, g_out, uid_v, rows_v, sem):
        wid = lax.axis_index("s") * nc + lax.axis_index("c")
        base = wid * upw
        pltpu.sync_copy(users_h.at[pl.ds(base, upw)], uid_v)
        pltpu.async_copy(g_h.at[uid_v], rows_v, sem).wait()
        pltpu.sync_copy(rows_v, g_out.at[pl.ds(base, upw)])

    return sc_ufeat(g_tab, users)


def _sc_pool_call(item_tab, user_hist_flat):
    info = plsc.get_sparse_core_info()
    nc, ns, nl = info.num_cores, info.num_subcores, info.num_lanes
    nw = nc * ns
    upw = B // nw
    c0 = 128
    c1 = HIST - c0
    dpack = D_TOW // 2
    npair = dpack // nl
    nslots = 4

    mesh = plsc.VectorSubcoreMesh(core_axis_name="c", subcore_axis_name="s")

    @functools.partial(
        pl.kernel,
        out_type=jax.ShapeDtypeStruct((B, D_TOW), jnp.float32),
        mesh=mesh,
        compiler_params=pltpu.CompilerParams(use_tc_tiling_on_sc=False,
                                             needs_layout_passes=False),
        scratch_types=[
            pltpu.VMEM((upw * HIST,), jnp.int32),
            pltpu.VMEM((nslots, HIST, dpack), jnp.float32),
            pltpu.VMEM((upw, D_TOW), jnp.float32),
            pltpu.SemaphoreType.DMA,
            pltpu.SemaphoreType.DMA,
            pltpu.SemaphoreType.DMA,
            pltpu.SemaphoreType.DMA,
        ],
    )
    def sc_pool(item_h, hist_h, hist_out,
                idx_all, rows_v, out_all_v,
                sem0, sem1, sem2, sem3):
        wid = lax.axis_index("s") * nc + lax.axis_index("c")
        base = wid * upw
        sems = (sem0, sem1, sem2, sem3)

        pltpu.sync_copy(hist_h.at[pl.ds(base * HIST, upw * HIST)], idx_all)

        def copies(slot, u):
            ra = rows_v.at[slot, pl.ds(0, c0)]
            rb = rows_v.at[slot, pl.ds(c0, c1)]
            sem = sems[slot]
            return (
                pltpu.make_async_copy(
                    item_h.at[idx_all.at[pl.ds(u * HIST, c0)]], ra, sem),
                pltpu.make_async_copy(
                    item_h.at[idx_all.at[pl.ds(u * HIST + c0, c1)]], rb, sem),
            )

        def issue(slot, u):
            for c in copies(slot, u):
                c.start()

        def wait_acc(slot, u):
            for c in copies(slot, u):
                c.wait()

            def acc_row(r, acc):
                new = list(acc)
                for k in range(npair):
                    x = rows_v[slot, r, pl.ds(k * nl, nl)]
                    lo, hi = plsc.unpack(plsc.bitcast(x, jnp.bfloat16),
                                         format=plsc.PackFormat.INTERLEAVED)
                    new[k] = acc[k] + lo
                    new[npair + k] = acc[npair + k] + hi
                return tuple(new)

            zero = tuple(jnp.zeros((nl,), jnp.float32)
                         for _ in range(2 * npair))
            acc = lax.fori_loop(0, HIST, acc_row, zero, unroll=2)
            for k in range(2 * npair):
                out_all_v[u, pl.ds(k * nl, nl)] = acc[k]

        for s in range(nslots):
            issue(s, s)

        def body(j, carry):
            u0 = nslots * j
            for s in range(nslots):
                wait_acc(s, u0 + s)

                @pl.when(u0 + s + nslots < upw)
                def _(s=s):
                    issue(s, u0 + s + nslots)

            return carry

        lax.fori_loop(0, upw // nslots, body, 0)
        pltpu.sync_copy(out_all_v, hist_out.at[pl.ds(base, upw)])

    return sc_pool(item_tab, user_hist_flat)


def _user_tower_body(g_ref, hs_ref, w1b_ref, w2_ref, b2_ref, out_ref):
    ug = lax.bitcast_convert_type(g_ref[...], jnp.uint32)
    g_lo = lax.bitcast_convert_type((ug & jnp.uint32(0xFFFF)) << 16,
                                    jnp.float32)
    g_hi = lax.bitcast_convert_type(ug & jnp.uint32(0xFFFF0000), jnp.float32)
    m = jnp.dot(hs_ref[...], w1b_ref[...], preferred_element_type=jnp.float32)
    h1 = jnp.maximum(g_lo + m[:, :D_HID // 2], 0.0).astype(jnp.bfloat16)
    h2 = jnp.maximum(g_hi + m[:, D_HID // 2:], 0.0).astype(jnp.bfloat16)
    e = jnp.dot(h1, w2_ref[:D_HID // 2], preferred_element_type=jnp.float32)
    e += jnp.dot(h2, w2_ref[D_HID // 2:], preferred_element_type=jnp.float32)
    e += b2_ref[...]
    n = jnp.sqrt(jnp.sum(e * e, axis=1, keepdims=True))
    out_ref[...] = e / jnp.maximum(n, 1e-12)


def _user_tower(g_rows, hist_sum, w1b, w2, b2):
    blk = 2048
    grid = B // blk
    return pl.pallas_call(
        _user_tower_body,
        grid=(grid,),
        in_specs=[
            pl.BlockSpec((blk, D_HID // 2), lambda i: (i, 0)),
            pl.BlockSpec((blk, D_TOW), lambda i: (i, 0)),
            pl.BlockSpec((D_TOW, D_HID), lambda i: (0, 0)),
            pl.BlockSpec((D_HID, D_TOW), lambda i: (0, 0)),
            pl.BlockSpec((1, D_TOW), lambda i: (0, 0)),
        ],
        out_specs=pl.BlockSpec((blk, D_TOW), lambda i: (i, 0)),
        out_shape=jax.ShapeDtypeStruct((B, D_TOW), jnp.float32),
    )(g_rows, hist_sum, w1b, w2, b2)


def kernel(item_text_emb, gcn_item_emb, gcn_user_emb, users, user_hist,
           Wi1, bi1, Wi2, bi2, Wu1, bu1, Wu2, bu2):
    wi1a = Wi1[:D_TEXT].astype(jnp.bfloat16)
    wi1b = Wi1[D_TEXT:].astype(jnp.bfloat16)
    item_emb = _item_tower(item_text_emb, gcn_item_emb.T, wi1a, wi1b,
                           bi1.reshape(1, -1), Wi2.astype(jnp.bfloat16),
                           bi2.reshape(1, -1))
    item_tab = item_emb.reshape(2 * N_ITEMS, D_TOW // 2)

    users_i = users.astype(jnp.int32)
    hist_i = (user_hist.astype(jnp.int32) * 2).reshape(-1)
    hist_sum = _sc_pool_call(item_tab, hist_i)

    g_tab = _user_pre(gcn_user_emb.T, Wu1[:D_GCN].astype(jnp.bfloat16),
                      bu1.reshape(1, -1))
    g_rows = _sc_ufeat_call(g_tab, users_i)

    wu1b = Wu1[D_GCN:] * (1.0 / HIST)
    return _user_tower(g_rows, hist_sum, wu1b, Wu2, bu2.reshape(1, -1))


# speedup vs baseline: 13.1245x; 1.1776x over previous
import functools

import jax
import jax.numpy as jnp
import numpy as np
from jax import lax
from jax.experimental import pallas as pl
from jax.experimental.pallas import tpu as pltpu
from jax.experimental.pallas import tpu_sc as plsc

N_ITEMS = 100000
N_USERS = 100000
B = 4096
HIST = 200
D_TEXT = 128
D_GCN = 64
D_HID = 256
D_TOW = 128

ITEM_BLOCK = 6400


def _item_tower_body(xt_ref, xgt_ref, w1a_ref, w1b_ref, b1_ref, w2_ref, b2_ref,
                     out_ref):
    xt = xt_ref[...].astype(jnp.bfloat16)
    xgt = xgt_ref[...].astype(jnp.bfloat16)
    h = jnp.dot(xt, w1a_ref[...], preferred_element_type=jnp.float32)
    h += lax.dot_general(xgt, w1b_ref[...], (((0,), (0,)), ((), ())),
                         preferred_element_type=jnp.float32)
    h = jnp.maximum(h + b1_ref[...], 0.0).astype(jnp.bfloat16)
    e = jnp.dot(h, w2_ref[...], preferred_element_type=jnp.float32) + b2_ref[...]
    eb = e.astype(jnp.bfloat16)
    n2 = jnp.dot(eb * eb, jnp.ones((D_TOW, 1), jnp.bfloat16),
                 preferred_element_type=jnp.float32)
    inv = lax.rsqrt(jnp.maximum(n2, 1e-24)).astype(jnp.bfloat16)
    enb = eb * inv
    lo = lax.bitcast_convert_type(
        enb[:, :D_TOW // 2].astype(jnp.float32), jnp.uint32)
    hi = lax.bitcast_convert_type(
        enb[:, D_TOW // 2:].astype(jnp.float32), jnp.uint32)
    out_ref[:, :D_TOW // 2] = lax.bitcast_convert_type(
        (lo >> 16) | hi, jnp.float32)


def _item_tower(item_text_emb, gcn_item_emb_t, w1a, w1b, b1, w2, b2):
    grid = (N_ITEMS + ITEM_BLOCK - 1) // ITEM_BLOCK
    return pl.pallas_call(
        _item_tower_body,
        grid=(grid,),
        in_specs=[
            pl.BlockSpec((ITEM_BLOCK, D_TEXT), lambda i: (i, 0)),
            pl.BlockSpec((D_GCN, ITEM_BLOCK), lambda i: (0, i)),
            pl.BlockSpec((D_TEXT, D_HID), lambda i: (0, 0)),
            pl.BlockSpec((D_GCN, D_HID), lambda i: (0, 0)),
            pl.BlockSpec((1, D_HID), lambda i: (0, 0)),
            pl.BlockSpec((D_HID, D_TOW), lambda i: (0, 0)),
            pl.BlockSpec((1, D_TOW), lambda i: (0, 0)),
        ],
        out_specs=pl.BlockSpec((ITEM_BLOCK, D_TOW), lambda i: (i, 0)),
        out_shape=jax.ShapeDtypeStruct((N_ITEMS, D_TOW), jnp.float32),
    )(item_text_emb, gcn_item_emb_t, w1a, w1b, b1, w2, b2)


def _user_pre_body(xgt_ref, w_ref, b_ref, out_ref):
    xgt = xgt_ref[...].astype(jnp.bfloat16)
    g = lax.dot_general(xgt, w_ref[...], (((0,), (0,)), ((), ())),
                        preferred_element_type=jnp.float32) + b_ref[...]
    gb = lax.bitcast_convert_type(g.astype(jnp.bfloat16), jnp.uint16)
    lo = gb[:, :D_HID // 2].astype(jnp.uint32)
    hi = gb[:, D_HID // 2:].astype(jnp.uint32)
    out_ref[...] = lax.bitcast_convert_type(lo | (hi << 16), jnp.float32)


def _user_pre(gcn_user_emb_t, w1a, b1):
    blk = 12800
    grid = (N_USERS + blk - 1) // blk
    return pl.pallas_call(
        _user_pre_body,
        grid=(grid,),
        in_specs=[
            pl.BlockSpec((D_GCN, blk), lambda i: (0, i)),
            pl.BlockSpec((D_GCN, D_HID), lambda i: (0, 0)),
            pl.BlockSpec((1, D_HID), lambda i: (0, 0)),
        ],
        out_specs=pl.BlockSpec((blk, D_HID // 2), lambda i: (i, 0)),
        out_shape=jax.ShapeDtypeStruct((N_USERS, D_HID // 2), jnp.float32),
    )(gcn_user_emb_t, w1a, b1)


def _sc_ufeat_call(g_tab, users):
    info = plsc.get_sparse_core_info()
    nc, ns, nl = info.num_cores, info.num_subcores, info.num_lanes
    nw = nc * ns
    upw = B // nw

    mesh = plsc.VectorSubcoreMesh(core_axis_name="c", subcore_axis_name="s")

    @functools.partial(
        pl.kernel,
        out_type=jax.ShapeDtypeStruct((B, D_HID // 2), jnp.float32),
        mesh=mesh,
        compiler_params=pltpu.CompilerParams(use_tc_tiling_on_sc=False),
        scratch_types=[
            pltpu.VMEM((upw,), jnp.int32),
            pltpu.VMEM((upw, D_HID // 2), jnp.float32),
            pltpu.SemaphoreType.DMA,
        ],
    )
    def sc_ufeat(g_h, users_h, g_out, uid_v, rows_v, sem):
        wid = lax.axis_index("s") * nc + lax.axis_index("c")
        base = wid * upw
        pltpu.sync_copy(users_h.at[pl.ds(base, upw)], uid_v)
        pltpu.async_copy(g_h.at[uid_v], rows_v, sem).wait()
        pltpu.sync_copy(rows_v, g_out.at[pl.ds(base, upw)])

    return sc_ufeat(g_tab, users)


def _sc_pool_call(item_tab, user_hist_flat):
    info = plsc.get_sparse_core_info()
    nc, ns, nl = info.num_cores, info.num_subcores, info.num_lanes
    nw = nc * ns
    upw = B // nw
    c0 = 128
    c1 = HIST - c0
    dpack = D_TOW // 2
    npair = dpack // nl
    nslots = 4

    mesh = plsc.VectorSubcoreMesh(core_axis_name="c", subcore_axis_name="s")

    @functools.partial(
        pl.kernel,
        out_type=jax.ShapeDtypeStruct((B, D_TOW), jnp.float32),
        mesh=mesh,
        compiler_params=pltpu.CompilerParams(use_tc_tiling_on_sc=False,
                                             needs_layout_passes=False),
        scratch_types=[
            pltpu.VMEM((upw * HIST,), jnp.int32),
            pltpu.VMEM((nslots, HIST, dpack), jnp.float32),
            pltpu.VMEM((upw, D_TOW), jnp.float32),
            pltpu.SemaphoreType.DMA,
            pltpu.SemaphoreType.DMA,
            pltpu.SemaphoreType.DMA,
            pltpu.SemaphoreType.DMA,
        ],
    )
    def sc_pool(item_h, hist_h, hist_out,
                idx_all, rows_v, out_all_v,
                sem0, sem1, sem2, sem3):
        wid = lax.axis_index("s") * nc + lax.axis_index("c")
        base = wid * upw
        sems = (sem0, sem1, sem2, sem3)

        pltpu.sync_copy(hist_h.at[pl.ds(base * HIST, upw * HIST)], idx_all)

        def copies(slot, u):
            ra = rows_v.at[slot, pl.ds(0, c0)]
            rb = rows_v.at[slot, pl.ds(c0, c1)]
            sem = sems[slot]
            return (
                pltpu.make_async_copy(
                    item_h.at[idx_all.at[pl.ds(u * HIST, c0)]], ra, sem),
                pltpu.make_async_copy(
                    item_h.at[idx_all.at[pl.ds(u * HIST + c0, c1)]], rb, sem),
            )

        def issue(slot, u):
            for c in copies(slot, u):
                c.start()

        def wait_acc(slot, u):
            for c in copies(slot, u):
                c.wait()

            def acc_row(r, acc):
                new = list(acc)
                for k in range(npair):
                    x = rows_v[slot, r, pl.ds(k * nl, nl)]
                    lo, hi = plsc.unpack(plsc.bitcast(x, jnp.bfloat16),
                                         format=plsc.PackFormat.INTERLEAVED)
                    new[k] = acc[k] + lo
                    new[npair + k] = acc[npair + k] + hi
                return tuple(new)

            zero = tuple(jnp.zeros((nl,), jnp.float32)
                         for _ in range(2 * npair))
            acc = lax.fori_loop(0, HIST, acc_row, zero, unroll=2)
            for k in range(2 * npair):
                out_all_v[u, pl.ds(k * nl, nl)] = acc[k]

        for s in range(nslots):
            issue(s, s)

        def body(j, carry):
            u0 = nslots * j
            for s in range(nslots):
                wait_acc(s, u0 + s)

                @pl.when(u0 + s + nslots < upw)
                def _(s=s):
                    issue(s, u0 + s + nslots)

            return carry

        lax.fori_loop(0, upw // nslots, body, 0)
        pltpu.sync_copy(out_all_v, hist_out.at[pl.ds(base, upw)])

    return sc_pool(item_tab, user_hist_flat)


def _user_tower_body(g_ref, hs_ref, w1b_ref, w2_ref, b2_ref, out_ref):
    ug = lax.bitcast_convert_type(g_ref[...], jnp.uint32)
    g_lo = lax.bitcast_convert_type((ug & jnp.uint32(0xFFFF)) << 16,
                                    jnp.float32)
    g_hi = lax.bitcast_convert_type(ug & jnp.uint32(0xFFFF0000), jnp.float32)
    m = jnp.dot(hs_ref[...], w1b_ref[...], preferred_element_type=jnp.float32)
    h1 = jnp.maximum(g_lo + m[:, :D_HID // 2], 0.0).astype(jnp.bfloat16)
    h2 = jnp.maximum(g_hi + m[:, D_HID // 2:], 0.0).astype(jnp.bfloat16)
    e = jnp.dot(h1, w2_ref[:D_HID // 2], preferred_element_type=jnp.float32)
    e += jnp.dot(h2, w2_ref[D_HID // 2:], preferred_element_type=jnp.float32)
    e += b2_ref[...]
    n = jnp.sqrt(jnp.sum(e * e, axis=1, keepdims=True))
    out_ref[...] = e / jnp.maximum(n, 1e-12)


def _user_tower(g_rows, hist_sum, w1b, w2, b2):
    blk = 2048
    grid = B // blk
    return pl.pallas_call(
        _user_tower_body,
        grid=(grid,),
        in_specs=[
            pl.BlockSpec((blk, D_HID // 2), lambda i: (i, 0)),
            pl.BlockSpec((blk, D_TOW), lambda i: (i, 0)),
            pl.BlockSpec((D_TOW, D_HID), lambda i: (0, 0)),
            pl.BlockSpec((D_HID, D_TOW), lambda i: (0, 0)),
            pl.BlockSpec((1, D_TOW), lambda i: (0, 0)),
        ],
        out_specs=pl.BlockSpec((blk, D_TOW), lambda i: (i, 0)),
        out_shape=jax.ShapeDtypeStruct((B, D_TOW), jnp.float32),
    )(g_rows, hist_sum, w1b, w2, b2)


def kernel(item_text_emb, gcn_item_emb, gcn_user_emb, users, user_hist,
           Wi1, bi1, Wi2, bi2, Wu1, bu1, Wu2, bu2):
    wi1a = Wi1[:D_TEXT].astype(jnp.bfloat16)
    wi1b = Wi1[D_TEXT:].astype(jnp.bfloat16)
    item_emb = _item_tower(item_text_emb, gcn_item_emb.T, wi1a, wi1b,
                           bi1.reshape(1, -1), Wi2.astype(jnp.bfloat16),
                           bi2.reshape(1, -1))
    item_tab = item_emb.reshape(2 * N_ITEMS, D_TOW // 2)

    users_i = users.astype(jnp.int32)
    hist_i = (user_hist.astype(jnp.int32) * 2).reshape(-1)
    hist_sum = _sc_pool_call(item_tab, hist_i)

    g_tab = _user_pre(gcn_user_emb.T, Wu1[:D_GCN].astype(jnp.bfloat16),
                      bu1.reshape(1, -1))
    g_rows = _sc_ufeat_call(g_tab, users_i)

    wu1b = Wu1[D_GCN:] * (1.0 / HIST)
    return _user_tower(g_rows, hist_sum, wu1b, Wu2, bu2.reshape(1, -1))
